# baseline (speedup 1.0000x reference)
import functools
import jax
import jax.numpy as jnp
from jax.experimental import pallas as pl
from jax.experimental.pallas import tpu as pltpu

_EMBED = 2048
_STEPS = 8


_NBUF = 4


def _body(nblk, C, x_hbm, w_ref, b_ref, rw_ref, sp_ref, cnt_ref, ent_ref,
          exp_ref, xb, sem):
    i = pl.program_id(0)
    slot = jax.lax.rem(i, _NBUF)

    def _start(chunk, buf):
        pltpu.make_async_copy(
            x_hbm.at[pl.ds(chunk * C, C), :], xb.at[buf], sem.at[buf]).start()

    def _wait(chunk, buf):
        pltpu.make_async_copy(
            x_hbm.at[pl.ds(chunk * C, C), :], xb.at[buf], sem.at[buf]).wait()

    @pl.when(i == 0)
    def _prime():
        for j in range(_NBUF - 1):
            if j < nblk:
                _start(j, j)

    nxt_chunk = i + _NBUF - 1
    nxt_slot = jax.lax.rem(nxt_chunk, _NBUF)

    @pl.when(nxt_chunk < nblk)
    def _prefetch():
        _start(nxt_chunk, nxt_slot)

    _wait(i, slot)

    logits = jax.lax.dot_general(
        xb[slot], w_ref[...], (((1,), (1,)), ((), ())),
        preferred_element_type=jnp.float32,
    ) + b_ref[...]
    m = jnp.max(logits, axis=1, keepdims=True)
    l2 = jnp.clip(logits - m, -50.0, 50.0)
    s = l2 / (1.0 + 1e-8)
    e = jnp.exp(s)
    z = jnp.sum(e, axis=1, keepdims=True)
    p = e / z
    sp_ref[...] = p

    pmax = 1.0 / z
    t = (p == pmax).astype(jnp.float32)
    iota_c = jax.lax.broadcasted_iota(jnp.int32, (_STEPS, _STEPS), 0)
    iota_r = jax.lax.broadcasted_iota(jnp.int32, (_STEPS, _STEPS), 1)
    ltri = (iota_c < iota_r).astype(jnp.float32)
    excl = jax.lax.dot_general(t, ltri, (((1,), (0,)), ((), ())),
                               preferred_element_type=jnp.float32)
    rw = jnp.where(excl < 0.5, t, 0.0)
    rw_ref[...] = rw

    cnt_part = jnp.sum(rw, axis=0, keepdims=True)
    ent_col = jnp.sum(p * jnp.log(p + 1e-8), axis=0, keepdims=True)
    ent_part = -jnp.sum(ent_col, axis=1, keepdims=True)
    step_f = jax.lax.broadcasted_iota(jnp.int32, (1, _STEPS), 1).astype(jnp.float32)
    p_col = jnp.sum(p, axis=0, keepdims=True)
    exp_part = jnp.sum(p_col * step_f, axis=1, keepdims=True)

    @pl.when(i == 0)
    def _init():
        cnt_ref[...] = jnp.zeros_like(cnt_ref)
        ent_ref[...] = jnp.zeros_like(ent_ref)
        exp_ref[...] = jnp.zeros_like(exp_ref)

    cnt_ref[...] += cnt_part
    ent_ref[...] += ent_part
    exp_ref[...] += exp_part

    @pl.when(i == nblk - 1)
    def _finalize():
        ntok = jnp.float32(nblk) * jnp.float32(C)
        ent_ref[...] = jnp.clip(ent_ref[...] / ntok, 0.0, 20.0)
        exp_ref[...] = exp_ref[...] / ntok


def kernel(x, W, b):
    bsz, seqlen, d = x.shape
    ntok = bsz * seqlen
    x_flat = x.reshape(ntok, d)
    b2 = b.reshape(1, _STEPS)
    C = 1024
    nblk = ntok // C

    body = functools.partial(_body, nblk, C)
    f32 = jnp.float32
    rw, sp, cnt, ent, exp_steps = pl.pallas_call(
        body,
        grid=(nblk,),
        in_specs=[
            pl.BlockSpec(memory_space=pl.ANY),
            pl.BlockSpec((_STEPS, d), lambda i: (0, 0)),
            pl.BlockSpec((1, _STEPS), lambda i: (0, 0)),
        ],
        out_specs=[
            pl.BlockSpec((C, _STEPS), lambda i: (i, 0)),
            pl.BlockSpec((C, _STEPS), lambda i: (i, 0)),
            pl.BlockSpec((1, _STEPS), lambda i: (0, 0)),
            pl.BlockSpec((1, 1), lambda i: (0, 0)),
            pl.BlockSpec((1, 1), lambda i: (0, 0)),
        ],
        out_shape=[
            jax.ShapeDtypeStruct((ntok, _STEPS), f32),
            jax.ShapeDtypeStruct((ntok, _STEPS), f32),
            jax.ShapeDtypeStruct((1, _STEPS), f32),
            jax.ShapeDtypeStruct((1, 1), f32),
            jax.ShapeDtypeStruct((1, 1), f32),
        ],
        scratch_shapes=[
            pltpu.VMEM((_NBUF, C, d), f32),
            pltpu.SemaphoreType.DMA((_NBUF,)),
        ],
    )(x_flat, W, b2)

    return (
        rw.reshape(bsz, seqlen, _STEPS),
        sp.reshape(bsz, seqlen, _STEPS),
        ent[0, 0],
        exp_steps[0, 0],
        cnt[0],
    )
